# baseline (speedup 1.0000x reference)
import functools

import jax
import jax.numpy as jnp
from jax import lax
from jax.experimental import pallas as pl
from jax.experimental.pallas import tpu as pltpu
from jax.experimental.pallas import tpu_sc as plsc

_VOCAB = 100000
_EMBED = 64
_B = 4096
_S = 50
_NC = 2
_NS = 16
_NW = _NC * _NS
_EPW = _EMBED // _NW
_L = 16

_LOG1P_C = (
    9.999999954e-01, -4.999990405e-01, 3.333000497e-01, -2.495456563e-01,
    1.967814349e-01, -1.531192208e-01, 1.061434342e-01, -5.706481085e-02,
    1.990741108e-02, -3.256419582e-03,
)


def _softplus16(x):
    t = jnp.exp(-jnp.abs(x))
    q = jnp.full((_L,), _LOG1P_C[-1], jnp.float32)
    for c in _LOG1P_C[-2::-1]:
        q = q * t + c
    return jnp.maximum(x, 0.0) + t * q


def _make_tgather():
    mesh = plsc.VectorSubcoreMesh(
        core_axis_name="c", subcore_axis_name="s",
        num_cores=_NC, num_subcores=_NS)

    @functools.partial(
        pl.kernel,
        out_type=jax.ShapeDtypeStruct((_S, _EMBED, _B), jnp.float32),
        mesh=mesh,
        compiler_params=pltpu.CompilerParams(needs_layout_passes=False),
        scratch_types=[
            pltpu.VMEM((_VOCAB,), jnp.float32),
            pltpu.VMEM((_B,), jnp.int32),
            pltpu.VMEM((_B,), jnp.float32),
            pltpu.SemaphoreType.DMA,
        ],
    )
    def tgather(rawT_hbm, idxT_hbm, out_hbm, tbl_v, idx_v, out_v, sem):
        wid = lax.axis_index("s") * _NC + lax.axis_index("c")

        @pl.loop(0, _EPW)
        def _erow(r):
            e = wid * _EPW + r
            pltpu.sync_copy(rawT_hbm.at[e], tbl_v)

            @pl.loop(0, _VOCAB // _L, unroll=8)
            def _sp(k):
                tbl_v[pl.ds(k * _L, _L)] = _softplus16(tbl_v[pl.ds(k * _L, _L)])

            @pl.loop(0, _S)
            def _slab(s):
                pltpu.sync_copy(idxT_hbm.at[s], idx_v)

                @pl.loop(0, _B // _L, unroll=8)
                def _g(k):
                    iv = idx_v[pl.ds(k * _L, _L)]
                    out_v[pl.ds(k * _L, _L)] = plsc.load_gather(tbl_v, [iv])

                pltpu.sync_copy(out_v, out_hbm.at[s, e])

    return tgather


def kernel(idx, raw):
    rawT = jnp.transpose(raw)
    idxT = jnp.transpose(idx.astype(jnp.int32))
    outT = _make_tgather()(rawT, idxT)
    return jnp.transpose(outT, (2, 0, 1))


# speedup vs baseline: 1.3221x; 1.3221x over previous
import functools

import jax
import jax.numpy as jnp
from jax import lax
from jax.experimental import pallas as pl
from jax.experimental.pallas import tpu as pltpu
from jax.experimental.pallas import tpu_sc as plsc

_VOCAB = 100000
_EMBED = 64
_B = 4096
_S = 50
_NC = 2
_NS = 16
_NW = _NC * _NS
_EPW = _EMBED // _NW
_L = 16

_LOG1P_C = (
    9.999999954e-01, -4.999990405e-01, 3.333000497e-01, -2.495456563e-01,
    1.967814349e-01, -1.531192208e-01, 1.061434342e-01, -5.706481085e-02,
    1.990741108e-02, -3.256419582e-03,
)


def _softplus16(x):
    t = jnp.exp(-jnp.abs(x))
    q = jnp.full((_L,), _LOG1P_C[-1], jnp.float32)
    for c in _LOG1P_C[-2::-1]:
        q = q * t + c
    return jnp.maximum(x, 0.0) + t * q


def _make_tgather():
    mesh = plsc.VectorSubcoreMesh(
        core_axis_name="c", subcore_axis_name="s",
        num_cores=_NC, num_subcores=_NS)

    @functools.partial(
        pl.kernel,
        out_type=jax.ShapeDtypeStruct((_S, _EMBED, _B), jnp.float32),
        mesh=mesh,
        compiler_params=pltpu.CompilerParams(needs_layout_passes=False),
        scratch_types=[
            pltpu.VMEM((_VOCAB,), jnp.float32),
            pltpu.VMEM((_B,), jnp.int32),
            pltpu.VMEM((_B,), jnp.int32),
            pltpu.VMEM((_B,), jnp.float32),
            pltpu.VMEM((_B,), jnp.float32),
            pltpu.SemaphoreType.DMA,
            pltpu.SemaphoreType.DMA,
            pltpu.SemaphoreType.DMA,
            pltpu.SemaphoreType.DMA,
        ],
    )
    def tgather(rawT_hbm, idxT_hbm, out_hbm, tbl_v, idx_a, idx_b,
                out_a, out_b, sem_ia, sem_ib, sem_oa, sem_ob):
        wid = lax.axis_index("s") * _NC + lax.axis_index("c")

        def _gather_slab(idx_v, out_v):
            @pl.loop(0, _B // _L, unroll=8)
            def _g(k):
                iv = idx_v[pl.ds(k * _L, _L)]
                out_v[pl.ds(k * _L, _L)] = plsc.load_gather(tbl_v, [iv])

        @pl.loop(0, _EPW)
        def _erow(r):
            e = wid * _EPW + r
            pltpu.async_copy(idxT_hbm.at[0], idx_a, sem_ia)
            pltpu.sync_copy(rawT_hbm.at[e], tbl_v)

            @pl.loop(0, _VOCAB // _L, unroll=8)
            def _sp(k):
                tbl_v[pl.ds(k * _L, _L)] = _softplus16(tbl_v[pl.ds(k * _L, _L)])

            @pl.loop(0, _S // 2)
            def _slab2(i):
                s0 = 2 * i
                s1 = s0 + 1
                pltpu.make_async_copy(idxT_hbm.at[s0], idx_a, sem_ia).wait()
                pltpu.async_copy(idxT_hbm.at[s1], idx_b, sem_ib)

                @pl.when(i > 0)
                def _():
                    pltpu.make_async_copy(out_a, out_hbm.at[s0, e], sem_oa).wait()

                _gather_slab(idx_a, out_a)
                pltpu.async_copy(out_a, out_hbm.at[s0, e], sem_oa)

                pltpu.make_async_copy(idxT_hbm.at[s1], idx_b, sem_ib).wait()

                @pl.when(s1 + 1 < _S)
                def _():
                    pltpu.async_copy(idxT_hbm.at[s1 + 1], idx_a, sem_ia)

                @pl.when(i > 0)
                def _():
                    pltpu.make_async_copy(out_b, out_hbm.at[s1, e], sem_ob).wait()

                _gather_slab(idx_b, out_b)
                pltpu.async_copy(out_b, out_hbm.at[s1, e], sem_ob)

            pltpu.make_async_copy(out_a, out_hbm.at[_S - 2, e], sem_oa).wait()
            pltpu.make_async_copy(out_b, out_hbm.at[_S - 1, e], sem_ob).wait()

    return tgather


def kernel(idx, raw):
    rawT = jnp.transpose(raw)
    idxT = jnp.transpose(idx.astype(jnp.int32))
    outT = _make_tgather()(rawT, idxT)
    return jnp.transpose(outT, (2, 0, 1))


# speedup vs baseline: 2.3188x; 1.7539x over previous
import functools

import jax
import jax.numpy as jnp
from jax import lax
from jax.experimental import pallas as pl
from jax.experimental.pallas import tpu as pltpu
from jax.experimental.pallas import tpu_sc as plsc

_VOCAB = 100000
_EMBED = 64
_B = 4096
_S = 50
_NC = 2
_NS = 16
_NW = _NC * _NS
_EPW = _EMBED // _NW
_L = 16

_LOG1P_C = (
    9.999987654e-01, -4.998719626e-01, 3.311208416e-01, -2.351495691e-01,
    1.494358462e-01, -6.658882788e-02, 1.420299358e-02,
)


def _softplus16(x):
    t = jnp.exp(-jnp.abs(x))
    q = jnp.full((_L,), _LOG1P_C[-1], jnp.float32)
    for c in _LOG1P_C[-2::-1]:
        q = q * t + c
    return jnp.maximum(x, 0.0) + t * q


def _make_tgather():
    mesh = plsc.VectorSubcoreMesh(
        core_axis_name="c", subcore_axis_name="s",
        num_cores=_NC, num_subcores=_NS)

    @functools.partial(
        pl.kernel,
        out_type=jax.ShapeDtypeStruct((_S, _EMBED, _B), jnp.float32),
        mesh=mesh,
        compiler_params=pltpu.CompilerParams(needs_layout_passes=False),
        scratch_types=[
            pltpu.VMEM((_VOCAB,), jnp.float32),
            pltpu.VMEM((_B,), jnp.int32),
            pltpu.VMEM((_B,), jnp.int32),
            pltpu.VMEM((_B,), jnp.float32),
            pltpu.VMEM((_B,), jnp.float32),
            pltpu.SemaphoreType.DMA,
            pltpu.SemaphoreType.DMA,
            pltpu.SemaphoreType.DMA,
            pltpu.SemaphoreType.DMA,
        ],
    )
    def tgather(rawT_hbm, idxT_hbm, out_hbm, tbl_v, idx_a, idx_b,
                out_a, out_b, sem_ia, sem_ib, sem_oa, sem_ob):
        wid = lax.axis_index("s") * _NC + lax.axis_index("c")

        _G = 8

        def _gather_slab(idx_v, out_v):
            @pl.loop(0, _B // (_L * _G))
            def _g(g):
                base = g * (_L * _G)
                ivs = [idx_v[pl.ds(base + j * _L, _L)] for j in range(_G)]
                vals = [plsc.load_gather(tbl_v, [iv]) for iv in ivs]
                for j in range(_G):
                    out_v[pl.ds(base + j * _L, _L)] = vals[j]

        @pl.loop(0, _EPW)
        def _erow(r):
            e = wid * _EPW + r
            pltpu.async_copy(idxT_hbm.at[0], idx_a, sem_ia)
            with jax.named_scope("tbl_load"):
                pltpu.sync_copy(rawT_hbm.at[e], tbl_v)

            with jax.named_scope("softplus"):
                @pl.loop(0, _VOCAB // (_L * _G))
                def _sp(g):
                    base = g * (_L * _G)
                    xs = [tbl_v[pl.ds(base + j * _L, _L)] for j in range(_G)]
                    ys = [_softplus16(x) for x in xs]
                    for j in range(_G):
                        tbl_v[pl.ds(base + j * _L, _L)] = ys[j]

                tail = (_VOCAB // (_L * _G)) * (_L * _G)
                for k in range(tail, _VOCAB, _L):
                    tbl_v[pl.ds(k, _L)] = _softplus16(tbl_v[pl.ds(k, _L)])

            @pl.loop(0, _S // 2)
            def _slab2(i):
                s0 = 2 * i
                s1 = s0 + 1
                pltpu.make_async_copy(idxT_hbm.at[s0], idx_a, sem_ia).wait()
                pltpu.async_copy(idxT_hbm.at[s1], idx_b, sem_ib)

                @pl.when(i > 0)
                def _():
                    pltpu.make_async_copy(out_a, out_hbm.at[s0, e], sem_oa).wait()

                _gather_slab(idx_a, out_a)
                pltpu.async_copy(out_a, out_hbm.at[s0, e], sem_oa)

                pltpu.make_async_copy(idxT_hbm.at[s1], idx_b, sem_ib).wait()

                @pl.when(s1 + 1 < _S)
                def _():
                    pltpu.async_copy(idxT_hbm.at[s1 + 1], idx_a, sem_ia)

                @pl.when(i > 0)
                def _():
                    pltpu.make_async_copy(out_b, out_hbm.at[s1, e], sem_ob).wait()

                _gather_slab(idx_b, out_b)
                pltpu.async_copy(out_b, out_hbm.at[s1, e], sem_ob)

            pltpu.make_async_copy(out_a, out_hbm.at[_S - 2, e], sem_oa).wait()
            pltpu.make_async_copy(out_b, out_hbm.at[_S - 1, e], sem_ob).wait()

    return tgather


def kernel(idx, raw):
    rawT = jnp.transpose(raw)
    idxT = jnp.transpose(idx.astype(jnp.int32))
    outT = _make_tgather()(rawT, idxT)
    return jnp.transpose(outT, (2, 0, 1))


# speedup vs baseline: 3.0140x; 1.2998x over previous
import functools

import jax
import jax.numpy as jnp
from jax import lax
from jax.experimental import pallas as pl
from jax.experimental.pallas import tpu as pltpu
from jax.experimental.pallas import tpu_sc as plsc

_VOCAB = 100000
_EMBED = 64
_B = 4096
_S = 50
_NC = 2
_NS = 16
_NW = _NC * _NS
_EPW = _EMBED // _NW
_L = 16
_G = 8
_NIB = 4

_LOG1P_C = (
    9.999450501e-01, -4.970314631e-01, 3.065628442e-01, -1.578400499e-01,
    4.155156826e-02,
)


def _softplus16(x):
    t = jnp.exp(jnp.minimum(x, -x))
    q = jnp.full((_L,), _LOG1P_C[-1], jnp.float32)
    for c in _LOG1P_C[-2::-1]:
        q = q * t + c
    return jnp.maximum(x, 0.0) + t * q


def _make_tgather():
    mesh = plsc.VectorSubcoreMesh(
        core_axis_name="c", subcore_axis_name="s",
        num_cores=_NC, num_subcores=_NS)

    @functools.partial(
        pl.kernel,
        out_type=jax.ShapeDtypeStruct((_S, _EMBED, _B), jnp.float32),
        mesh=mesh,
        compiler_params=pltpu.CompilerParams(needs_layout_passes=False),
        scratch_types=[
            pltpu.VMEM((_VOCAB,), jnp.float32),
            [pltpu.VMEM((_B,), jnp.int32) for _ in range(_NIB)],
            pltpu.VMEM((_B,), jnp.float32),
            pltpu.VMEM((_B,), jnp.float32),
            [pltpu.SemaphoreType.DMA for _ in range(_NIB)],
            pltpu.SemaphoreType.DMA,
            pltpu.SemaphoreType.DMA,
        ],
    )
    def tgather(rawT_hbm, idxT_hbm, out_hbm, tbl_v, idx_bufs,
                out_a, out_b, sem_idx, sem_oa, sem_ob):
        wid = lax.axis_index("s") * _NC + lax.axis_index("c")

        def _gather_slab(idx_v, out_v):
            @pl.loop(0, _B // (_L * _G))
            def _g(g):
                base = g * (_L * _G)
                ivs = [idx_v[pl.ds(base + j * _L, _L)] for j in range(_G)]
                vals = [plsc.load_gather(tbl_v, [iv]) for iv in ivs]
                for j in range(_G):
                    out_v[pl.ds(base + j * _L, _L)] = vals[j]

        @pl.loop(0, _EPW)
        def _erow(r):
            e = wid * _EPW + r
            for b in range(_NIB):
                pltpu.async_copy(idxT_hbm.at[b], idx_bufs[b], sem_idx[b])
            with jax.named_scope("tbl_load"):
                pltpu.sync_copy(rawT_hbm.at[e], tbl_v)

            with jax.named_scope("softplus"):
                @pl.loop(0, _VOCAB // (_L * _G))
                def _sp(g):
                    base = g * (_L * _G)
                    xs = [tbl_v[pl.ds(base + j * _L, _L)] for j in range(_G)]
                    ys = [_softplus16(x) for x in xs]
                    for j in range(_G):
                        tbl_v[pl.ds(base + j * _L, _L)] = ys[j]

                tail = (_VOCAB // (_L * _G)) * (_L * _G)
                for k in range(tail, _VOCAB, _L):
                    tbl_v[pl.ds(k, _L)] = _softplus16(tbl_v[pl.ds(k, _L)])

            def _slab(s, jj, first_use, refill):
                buf, sem = idx_bufs[jj], sem_idx[jj]
                out_v, sem_o = (out_a, sem_oa) if jj % 2 == 0 else (out_b, sem_ob)
                pltpu.make_async_copy(idxT_hbm.at[s], buf, sem).wait()
                if not first_use:
                    pltpu.make_async_copy(out_v, out_hbm.at[s, e], sem_o).wait()
                _gather_slab(buf, out_v)
                pltpu.async_copy(out_v, out_hbm.at[s, e], sem_o)
                if refill:
                    @pl.when(s + _NIB < _S)
                    def _():
                        pltpu.async_copy(idxT_hbm.at[s + _NIB], buf, sem)

            @pl.loop(0, (_S - 2) // _NIB)
            def _slab4(i):
                s_base = _NIB * i
                for jj in range(_NIB):
                    s = s_base + jj
                    if jj < 2:
                        @pl.when(s_base > 0)
                        def _(s=s, jj=jj):
                            _slab(s, jj, first_use=False, refill=True)

                        @pl.when(s_base == 0)
                        def _(s=s, jj=jj):
                            _slab(s, jj, first_use=True, refill=True)
                    else:
                        _slab(s, jj, first_use=False, refill=True)

            _slab(_S - 2, 0, first_use=False, refill=False)
            _slab(_S - 1, 1, first_use=False, refill=False)

            pltpu.make_async_copy(out_a, out_hbm.at[_S - 2, e], sem_oa).wait()
            pltpu.make_async_copy(out_b, out_hbm.at[_S - 1, e], sem_ob).wait()

    return tgather


def kernel(idx, raw):
    rawT = jnp.transpose(raw)
    idxT = jnp.transpose(idx.astype(jnp.int32))
    outT = _make_tgather()(rawT, idxT)
    return jnp.transpose(outT, (2, 0, 1))


# speedup vs baseline: 3.1821x; 1.0558x over previous
import functools

import jax
import jax.numpy as jnp
from jax import lax
from jax.experimental import pallas as pl
from jax.experimental.pallas import tpu as pltpu
from jax.experimental.pallas import tpu_sc as plsc

_VOCAB = 100000
_EMBED = 64
_B = 4096
_S = 50
_NC = 2
_NS = 16
_NW = _NC * _NS
_L = 16
_G = 8
_CB = 2048
_NFULL = _VOCAB // _CB
_TAILA = 1664
_TAILB = 32

_LOG1P_C = (
    9.999450501e-01, -4.970314631e-01, 3.065628442e-01, -1.578400499e-01,
    4.155156826e-02,
)


def _softplus16(x):
    t = jnp.exp(jnp.minimum(x, -x))
    q = jnp.full((_L,), _LOG1P_C[-1], jnp.float32)
    for c in _LOG1P_C[-2::-1]:
        q = q * t + c
    return jnp.maximum(x, 0.0) + t * q


def _pack16(a, b):
    return plsc.bitcast(
        plsc.pack(a, b, format=plsc.PackFormat.INTERLEAVED), jnp.float32)


def _unpack16(w):
    a, b = plsc.unpack(
        plsc.bitcast(w, jnp.bfloat16), format=plsc.PackFormat.INTERLEAVED)
    return a.astype(jnp.float32), b.astype(jnp.float32)


def _make_tgather():
    mesh = plsc.VectorSubcoreMesh(
        core_axis_name="c", subcore_axis_name="s",
        num_cores=_NC, num_subcores=_NS)

    @functools.partial(
        pl.kernel,
        out_type=jax.ShapeDtypeStruct((_S, _EMBED, _B), jnp.float32),
        mesh=mesh,
        compiler_params=pltpu.CompilerParams(needs_layout_passes=False),
        scratch_types=[
            pltpu.VMEM((_VOCAB,), jnp.float32),
            [pltpu.VMEM((_B,), jnp.int32) for _ in range(2)],
            [pltpu.VMEM((1, 2, _B), jnp.float32) for _ in range(2)],
            [pltpu.VMEM((1, _CB), jnp.float32) for _ in range(2)],
            pltpu.VMEM((1, _TAILA), jnp.float32),
            pltpu.VMEM((_TAILB,), jnp.float32),
            [pltpu.SemaphoreType.DMA for _ in range(2)],
            [pltpu.SemaphoreType.DMA for _ in range(2)],
            [pltpu.SemaphoreType.DMA for _ in range(2)],
        ],
    )
    def tgather(rawT_hbm, idxT_hbm, rawtail_hbm, out_hbm, tbl_v, idx_bufs,
                out_bufs, bbufs, tailbuf, tail2, sem_idx, sem_out, sem_b):
        wid = lax.axis_index("s") * _NC + lax.axis_index("c")
        e0 = wid * 2

        pltpu.async_copy(idxT_hbm.at[0], idx_bufs[0], sem_idx[0])
        pltpu.async_copy(idxT_hbm.at[1], idx_bufs[1], sem_idx[1])
        with jax.named_scope("tbl_load"):
            pltpu.sync_copy(rawT_hbm.at[e0], tbl_v)
        pltpu.async_copy(rawT_hbm.at[pl.ds(e0 + 1, 1), pl.ds(0, _CB)], bbufs[0], sem_b[0])
        pltpu.async_copy(rawT_hbm.at[pl.ds(e0 + 1, 1), pl.ds(_CB, _CB)], bbufs[1], sem_b[1])

        def _pack_chunk(cw, bbuf, nwords):
            @pl.loop(0, nwords // (_L * _G))
            def _pk(g):
                base = g * (_L * _G)
                offs = [base + j * _L for j in range(_G)]
                avs = [tbl_v[pl.ds(cw + o, _L)] for o in offs]
                bvs = [bbuf[0, pl.ds(o, _L)] for o in offs]
                pas = [_softplus16(a) for a in avs]
                pbs = [_softplus16(b) for b in bvs]
                pks = [_pack16(pa, pb) for pa, pb in zip(pas, pbs)]
                for j in range(_G):
                    tbl_v[pl.ds(cw + offs[j], _L)] = pks[j]

        with jax.named_scope("softplus_pack"):
            @pl.loop(0, _NFULL // 2)
            def _pair(p):
                for par in range(2):
                    c = 2 * p + par
                    bbuf, sem = bbufs[par], sem_b[par]
                    pltpu.make_async_copy(
                        rawT_hbm.at[pl.ds(e0 + 1, 1), pl.ds(0, _CB)], bbuf,
                        sem).wait()
                    _pack_chunk(c * _CB, bbuf, _CB)

                    @pl.when(c + 2 < _NFULL)
                    def _(c=c, bbuf=bbuf, sem=sem):
                        pltpu.async_copy(
                            rawT_hbm.at[pl.ds(e0 + 1, 1),
                                        pl.ds((c + 2) * _CB, _CB)],
                            bbuf, sem)

            pltpu.sync_copy(
                rawT_hbm.at[pl.ds(e0 + 1, 1), pl.ds(_NFULL * _CB, _TAILA)],
                tailbuf)
            _pack_chunk(_NFULL * _CB, tailbuf, _TAILA)
            pltpu.sync_copy(rawtail_hbm.at[e0 + 1], tail2)
            t2base = _NFULL * _CB + _TAILA
            for j in range(_TAILB // _L):
                a = tbl_v[pl.ds(t2base + j * _L, _L)]
                b = tail2[pl.ds(j * _L, _L)]
                tbl_v[pl.ds(t2base + j * _L, _L)] = _pack16(
                    _softplus16(a), _softplus16(b))

        def _gather_slab(s, bi):
            idx_v, out2, sem_o = idx_bufs[bi], out_bufs[bi], sem_out[bi]
            pltpu.make_async_copy(idxT_hbm.at[s], idx_v, sem_idx[bi]).wait()

            @pl.when(s >= 2)
            def _():
                pltpu.make_async_copy(
                    out2, out_hbm.at[pl.ds(s, 1), pl.ds(e0, 2)], sem_o).wait()

            @pl.loop(0, _B // (_L * _G))
            def _g(g):
                base = g * (_L * _G)
                ivs = [idx_v[pl.ds(base + j * _L, _L)] for j in range(_G)]
                ws = [plsc.load_gather(tbl_v, [iv]) for iv in ivs]
                abs_ = [_unpack16(w) for w in ws]
                for j in range(_G):
                    out2[0, 0, pl.ds(base + j * _L, _L)] = abs_[j][0]
                    out2[0, 1, pl.ds(base + j * _L, _L)] = abs_[j][1]

            pltpu.async_copy(out2, out_hbm.at[pl.ds(s, 1), pl.ds(e0, 2)], sem_o)

            @pl.when(s + 2 < _S)
            def _():
                pltpu.async_copy(idxT_hbm.at[s + 2], idx_v, sem_idx[bi])

        @pl.loop(0, _S // 2)
        def _slab2(i):
            _gather_slab(2 * i, 0)
            _gather_slab(2 * i + 1, 1)

        pltpu.make_async_copy(
            out_bufs[0], out_hbm.at[pl.ds(_S - 2, 1), pl.ds(e0, 2)], sem_out[0]).wait()
        pltpu.make_async_copy(
            out_bufs[1], out_hbm.at[pl.ds(_S - 1, 1), pl.ds(e0, 2)], sem_out[1]).wait()

    return tgather


def kernel(idx, raw):
    rawT = jnp.transpose(raw)
    idxT = jnp.transpose(idx.astype(jnp.int32))
    raw_tail = jnp.transpose(raw[_NFULL * _CB + _TAILA:, :])
    outT = _make_tgather()(rawT, idxT, raw_tail)
    return jnp.transpose(outT, (2, 0, 1))
